# baseline (speedup 1.0000x reference)
import functools

import jax
import jax.numpy as jnp
from jax import lax
from jax.experimental import pallas as pl
from jax.experimental.pallas import tpu as pltpu
from jax.experimental.pallas import tpu_sc as plsc

N_FIELDS = 26
VOCAB = 100000
EMBED_DIM = 64
BATCH = 4096
OUT_D = N_FIELDS * EMBED_DIM

_NC = 2
_NS = 16
_NW = _NC * _NS
_BPW = BATCH // _NW
_CB = 8
_CROWS = _CB * N_FIELDS
_NCHUNK = _BPW // _CB
_LANES = 16
_VEC = EMBED_DIM // _LANES


@functools.partial(
    pl.kernel,
    out_type=jax.ShapeDtypeStruct((BATCH, OUT_D), jnp.float32),
    mesh=plsc.VectorSubcoreMesh(core_axis_name="c", subcore_axis_name="s"),
    scratch_types=[
        pltpu.VMEM((_BPW, N_FIELDS), jnp.int32),
        pltpu.VMEM((_CROWS, EMBED_DIM), jnp.float32),
        pltpu.VMEM((_CROWS, EMBED_DIM), jnp.float32),
        pltpu.VMEM((_CB, OUT_D), jnp.float32),
        pltpu.VMEM((_CB, OUT_D), jnp.float32),
        pltpu.SemaphoreType.DMA,
        pltpu.SemaphoreType.DMA,
        pltpu.SemaphoreType.DMA,
        pltpu.SemaphoreType.DMA,
    ],
    compiler_params=pltpu.CompilerParams(use_tc_tiling_on_sc=True),
)
def _gather(x_hbm, tab_hbm, out_hbm, xbuf, r0, r1, ob0, ob1,
            g0, g1, o0, o1):
    wid = lax.axis_index("s") * _NC + lax.axis_index("c")
    bbase = wid * _BPW
    rbufs = (r0, r1)
    obufs = (ob0, ob1)
    gsems = (g0, g1)
    osems = (o0, o1)

    pltpu.sync_copy(x_hbm.at[pl.ds(bbase, _BPW)], xbuf)

    def fire(m, b):
        for bb in range(_CB):
            row = m * _CB + bb
            v0 = xbuf[row, pl.ds(0, _LANES)]
            v1 = xbuf[row, pl.ds(N_FIELDS - _LANES, _LANES)]
            for f in range(N_FIELDS):
                v = v0[f] if f < _LANES else v1[f - (N_FIELDS - _LANES)]
                r = bb * N_FIELDS + f
                pltpu.make_async_copy(
                    tab_hbm.at[pl.ds(v + f * VOCAB, 1)],
                    rbufs[b].at[pl.ds(r, 1)],
                    gsems[b]).start()

    def gwait(b):
        pltpu.make_async_copy(
            tab_hbm.at[pl.ds(0, _CROWS)], rbufs[b], gsems[b]).wait()

    def repack(b):
        for bb in range(_CB):
            for f in range(N_FIELDS):
                r = bb * N_FIELDS + f
                for k in range(_VEC):
                    obufs[b][bb, pl.ds(f * EMBED_DIM + k * _LANES, _LANES)] = (
                        rbufs[b][r, pl.ds(k * _LANES, _LANES)])

    def ostart(m, b):
        pltpu.make_async_copy(
            obufs[b], out_hbm.at[pl.ds(bbase + m * _CB, _CB)],
            osems[b]).start()

    def owait(b):
        pltpu.make_async_copy(
            obufs[b], out_hbm.at[pl.ds(bbase, _CB)], osems[b]).wait()

    fire(0, 0)
    fire(1, 1)
    for b in range(2):
        pltpu.make_async_copy(
            out_hbm.at[pl.ds(bbase, _CB)], obufs[b], osems[b]).start()

    def pipe_body(i, carry):
        for b in range(2):
            m = 2 * i + b
            gwait(b)
            owait(b)
            repack(b)
            ostart(m, b)
            fire(m + 2, b)
        return carry

    lax.fori_loop(0, _NCHUNK // 2 - 1, pipe_body, 0)

    for m in (_NCHUNK - 2, _NCHUNK - 1):
        b = m % 2
        gwait(b)
        owait(b)
        repack(b)
        ostart(m, b)
    owait(0)
    owait(1)


def kernel(x, tables):
    tab = tables.reshape(N_FIELDS * VOCAB, EMBED_DIM)
    return _gather(x.astype(jnp.int32), tab)
